# Initial kernel scaffold; baseline (speedup 1.0000x reference)
import jax
import jax.numpy as jnp
from jax.experimental import pallas as pl


def kernel(batch_basket, item_embedding):
    raise NotImplementedError("write your pallas kernel here")


# speedup vs baseline: 5.8853x; 5.8853x over previous
import functools

import jax
import jax.numpy as jnp
from jax import lax
from jax.experimental import pallas as pl
from jax.experimental.pallas import tpu as pltpu
from jax.experimental.pallas import tpu_sc as plsc

B, L, S = 1024, 50, 8
HIDDEN = 64

NC, NS = 2, 16
NW = NC * NS
IDX_PER_W = (B * L * S) // NW
G = 128
NG = IDX_PER_W // G
ROWS_PER_G = G // S
NB = 4
HV = HIDDEN // 16


def _body(idx_hbm, table_hbm, out_hbm, idx_v, gbuf, obuf, *sems):
    wid = lax.axis_index("s") * NC + lax.axis_index("c")
    out_base = wid * (IDX_PER_W // S)

    pltpu.sync_copy(idx_hbm.at[wid], idx_v)

    def gather(j, b):
        return pltpu.make_async_copy(
            table_hbm.at[idx_v.at[j]], gbuf.at[b], sems[b])

    for b in range(NB):
        gather(b, b).start()

    def group(g, carry):
        for b in range(NB):
            j = g * NB + b
            gather(j, b).wait()

            def row(r, carry):
                base = r * S
                for h in range(HV):
                    sl = pl.ds(h * 16, 16)
                    acc = gbuf[b, base, sl]
                    for s in range(1, S):
                        acc = acc + gbuf[b, base + s, sl]
                    obuf[r, sl] = acc * (1.0 / S)
                return carry

            lax.fori_loop(0, ROWS_PER_G, row, 0, unroll=4)
            pltpu.sync_copy(obuf, out_hbm.at[pl.ds(out_base + j * ROWS_PER_G,
                                                   ROWS_PER_G)])

            @pl.when(j + NB < NG)
            def _():
                gather(j + NB, b).start()
        return carry

    lax.fori_loop(0, NG // NB, group, 0)


@functools.partial(jax.jit, static_argnames=())
def _run(idx, table):
    kfn = pl.kernel(
        _body,
        out_type=jax.ShapeDtypeStruct((B * L, HIDDEN), jnp.float32),
        mesh=plsc.VectorSubcoreMesh(
            core_axis_name="c", subcore_axis_name="s",
            num_cores=NC, num_subcores=NS),
        scratch_types=[
            pltpu.VMEM((NG, G), jnp.int32),
            pltpu.VMEM((NB, G, HIDDEN), jnp.float32),
            pltpu.VMEM((ROWS_PER_G, HIDDEN), jnp.float32),
        ] + [pltpu.SemaphoreType.DMA] * NB,
        compiler_params=pltpu.CompilerParams(use_tc_tiling_on_sc=False),
    )
    return kfn(idx, table)


def kernel(batch_basket, item_embedding):
    idx = batch_basket.reshape(NW, NG, G)
    out = _run(idx, item_embedding)
    return out.reshape(B, L, HIDDEN)


# speedup vs baseline: 6.4882x; 1.1024x over previous
import functools

import jax
import jax.numpy as jnp
from jax import lax
from jax.experimental import pallas as pl
from jax.experimental.pallas import tpu as pltpu
from jax.experimental.pallas import tpu_sc as plsc

B, L, S = 1024, 50, 8
HIDDEN = 64

NC, NS = 2, 16
NW = NC * NS
B_PER_W = B // NW
IDX_PER_W = B_PER_W * L * S
RG = 10
G = RG * S
NG = IDX_PER_W // G
GPB = L // RG
NB = 8
HV = HIDDEN // 16


def _body(idx_hbm, table_hbm, out_hbm, idx_v, gbuf, obuf, osems, *sems):
    wid = lax.axis_index("s") * NC + lax.axis_index("c")
    b_base = wid * B_PER_W

    pltpu.sync_copy(idx_hbm.at[pl.ds(wid * IDX_PER_W, IDX_PER_W)], idx_v)

    def gather(j, b):
        return pltpu.make_async_copy(
            table_hbm.at[idx_v.at[pl.ds(j * G, G)]], gbuf.at[b], sems[b])

    def out_copy(j, ob):
        bb = j // GPB
        l0 = (j % GPB) * RG
        return pltpu.make_async_copy(
            obuf.at[ob], out_hbm.at[b_base + bb, pl.ds(l0, RG)], osems[ob])

    for b in range(NB):
        gather(b, b).start()

    def group(g, carry):
        for b in range(NB):
            j = g * NB + b
            ob = b % 2
            gather(j, b).wait()

            @pl.when(j >= 2)
            def _():
                out_copy(j - 2, ob).wait()

            def row(r, carry):
                base = r * S
                for h in range(HV):
                    sl = pl.ds(h * 16, 16)
                    t0 = gbuf[b, base + 0, sl] + gbuf[b, base + 1, sl]
                    t1 = gbuf[b, base + 2, sl] + gbuf[b, base + 3, sl]
                    t2 = gbuf[b, base + 4, sl] + gbuf[b, base + 5, sl]
                    t3 = gbuf[b, base + 6, sl] + gbuf[b, base + 7, sl]
                    obuf[ob, r, sl] = ((t0 + t1) + (t2 + t3)) * (1.0 / S)
                return carry

            lax.fori_loop(0, RG, row, 0, unroll=5)
            out_copy(j, ob).start()

            @pl.when(j + NB < NG)
            def _():
                gather(j + NB, b).start()
        return carry

    lax.fori_loop(0, NG // NB, group, 0)
    out_copy(NG - 2, 0).wait()
    out_copy(NG - 1, 1).wait()


@jax.jit
def _run(idx, table):
    kfn = pl.kernel(
        _body,
        out_type=jax.ShapeDtypeStruct((B, L, HIDDEN), jnp.float32),
        mesh=plsc.VectorSubcoreMesh(
            core_axis_name="c", subcore_axis_name="s",
            num_cores=NC, num_subcores=NS),
        scratch_types=[
            pltpu.VMEM((IDX_PER_W,), jnp.int32),
            pltpu.VMEM((NB, G, HIDDEN), jnp.float32),
            pltpu.VMEM((2, RG, HIDDEN), jnp.float32),
            [pltpu.SemaphoreType.DMA] * 2,
        ] + [pltpu.SemaphoreType.DMA] * NB,
        compiler_params=pltpu.CompilerParams(use_tc_tiling_on_sc=False),
    )
    return kfn(idx, table)


def kernel(batch_basket, item_embedding):
    return _run(batch_basket.reshape(-1), item_embedding)


# speedup vs baseline: 6.6836x; 1.0301x over previous
import functools

import jax
import jax.numpy as jnp
from jax import lax
from jax.experimental import pallas as pl
from jax.experimental.pallas import tpu as pltpu
from jax.experimental.pallas import tpu_sc as plsc

B, L, S = 1024, 50, 8
HIDDEN = 64

NC, NS = 2, 16
NW = NC * NS
B_PER_W = B // NW
IDX_PER_W = B_PER_W * L * S
RG = 10
G = RG * S
NG = IDX_PER_W // G
GPB = L // RG
NB = 8
HV = HIDDEN // 32


def _body(idx_hbm, table_hbm, out_hbm, idx_v, gbuf, obuf, osems, *sems):
    wid = lax.axis_index("s") * NC + lax.axis_index("c")
    b_base = wid * B_PER_W

    pltpu.sync_copy(idx_hbm.at[pl.ds(wid * IDX_PER_W, IDX_PER_W)], idx_v)

    def gather(j, b):
        return pltpu.make_async_copy(
            table_hbm.at[idx_v.at[pl.ds(j * G, G)]], gbuf.at[b], sems[b])

    def out_copy(j, ob):
        bb = j // GPB
        l0 = (j % GPB) * RG
        return pltpu.make_async_copy(
            obuf.at[ob], out_hbm.at[b_base + bb, pl.ds(l0, RG)], osems[ob])

    for b in range(NB):
        gather(b, b).start()

    def group(g, carry):
        for b in range(NB):
            j = g * NB + b
            ob = b % 2
            gather(j, b).wait()

            @pl.when(j >= 2)
            def _():
                out_copy(j - 2, ob).wait()

            def row(r, carry):
                base = r * S
                for h in range(HV):
                    sl = pl.ds(h * 32, 32)
                    t0 = gbuf[b, base + 0, sl] + gbuf[b, base + 1, sl]
                    t1 = gbuf[b, base + 2, sl] + gbuf[b, base + 3, sl]
                    t2 = gbuf[b, base + 4, sl] + gbuf[b, base + 5, sl]
                    t3 = gbuf[b, base + 6, sl] + gbuf[b, base + 7, sl]
                    obuf[ob, r, sl] = ((t0 + t1) + (t2 + t3)) * (1.0 / S)
                return carry

            lax.fori_loop(0, RG, row, 0, unroll=5)
            out_copy(j, ob).start()

            @pl.when(j + NB < NG)
            def _():
                gather(j + NB, b).start()
        return carry

    lax.fori_loop(0, NG // NB, group, 0)
    out_copy(NG - 2, 0).wait()
    out_copy(NG - 1, 1).wait()


@jax.jit
def _run(idx, table):
    kfn = pl.kernel(
        _body,
        out_type=jax.ShapeDtypeStruct((B, L, HIDDEN), jnp.bfloat16),
        mesh=plsc.VectorSubcoreMesh(
            core_axis_name="c", subcore_axis_name="s",
            num_cores=NC, num_subcores=NS),
        scratch_types=[
            pltpu.VMEM((IDX_PER_W,), jnp.int32),
            pltpu.VMEM((NB, G, HIDDEN), jnp.bfloat16),
            pltpu.VMEM((2, RG, HIDDEN), jnp.bfloat16),
            [pltpu.SemaphoreType.DMA] * 2,
        ] + [pltpu.SemaphoreType.DMA] * NB,
        compiler_params=pltpu.CompilerParams(use_tc_tiling_on_sc=False),
    )
    return kfn(idx, table)


def kernel(batch_basket, item_embedding):
    out = _run(batch_basket.reshape(-1),
               item_embedding.astype(jnp.bfloat16))
    return out.astype(jnp.float32)
